# Initial kernel scaffold; baseline (speedup 1.0000x reference)
import functools

import jax
import jax.numpy as jnp
from jax import lax
from jax.experimental import pallas as pl
from jax.experimental.pallas import tpu as pltpu
from jax.experimental.pallas import tpu_sc as plsc

H = 100
L = 16


def kernel(input_ids, table):
    B, S = input_ids.shape
    V, D = table.shape
    assert S == 2 * H and D == 2 * L

    ids3 = input_ids.reshape(B, 2, H).astype(jnp.int32)

    info = plsc.get_sparse_core_info()
    NC, NS = info.num_cores, info.num_subcores
    NW = NC * NS
    PB = B // NW

    mesh = plsc.VectorSubcoreMesh(core_axis_name="c", subcore_axis_name="s")

    @functools.partial(
        pl.kernel,
        mesh=mesh,
        out_type=jax.ShapeDtypeStruct((B, D), jnp.float32),
        scratch_types=[
            pltpu.VMEM((2, 2, H), jnp.int32),
            pltpu.VMEM((2, S, D), jnp.float32),
            pltpu.VMEM((PB, D), jnp.float32),
            pltpu.SemaphoreType.DMA,
            pltpu.SemaphoreType.DMA,
        ],
    )
    def _emb_mean(ids_hbm, table_hbm, out_hbm, idx_buf, rows_buf, out_stage,
                  sem0, sem1):
        wid = lax.axis_index("s") * NC + lax.axis_index("c")
        base = wid * PB
        sems = (sem0, sem1)

        def fetch(b, buf):
            pltpu.sync_copy(ids_hbm.at[base + b], idx_buf.at[buf])
            pltpu.make_async_copy(
                table_hbm.at[idx_buf.at[buf, 0]],
                rows_buf.at[buf, pl.ds(0, H)], sems[buf]).start()
            pltpu.make_async_copy(
                table_hbm.at[idx_buf.at[buf, 1]],
                rows_buf.at[buf, pl.ds(H, H)], sems[buf]).start()

        def drain(buf):
            pltpu.make_async_copy(
                table_hbm.at[idx_buf.at[buf, 0]],
                rows_buf.at[buf, pl.ds(0, H)], sems[buf]).wait()
            pltpu.make_async_copy(
                table_hbm.at[idx_buf.at[buf, 1]],
                rows_buf.at[buf, pl.ds(H, H)], sems[buf]).wait()

        def reduce(b, buf):
            def rbody(s8, accs):
                accs = list(accs)
                for j in range(8):
                    r = s8 * 8 + j
                    k = j % 4
                    accs[2 * k] = accs[2 * k] + rows_buf[buf, r, pl.ds(0, L)]
                    accs[2 * k + 1] = (
                        accs[2 * k + 1] + rows_buf[buf, r, pl.ds(L, L)])
                return tuple(accs)

            z = jnp.zeros((L,), jnp.float32)
            a = lax.fori_loop(0, S // 8, rbody, (z,) * 8)
            lo = (a[0] + a[2]) + (a[4] + a[6])
            hi = (a[1] + a[3]) + (a[5] + a[7])
            out_stage[b, pl.ds(0, L)] = lo * (1.0 / S)
            out_stage[b, pl.ds(L, L)] = hi * (1.0 / S)

        fetch(0, 0)

        def gbody(g, carry):
            fetch(2 * g + 1, 1)
            drain(0)
            reduce(2 * g, 0)

            @pl.when(2 * g + 2 < PB)
            def _():
                fetch(2 * g + 2, 0)

            drain(1)
            reduce(2 * g + 1, 1)
            return carry

        lax.fori_loop(0, PB // 2, gbody, 0)
        pltpu.sync_copy(out_stage, out_hbm.at[pl.ds(base, PB)])

    return _emb_mean(ids3, table)


# speedup vs baseline: 24.7023x; 24.7023x over previous
import functools

import jax
import jax.numpy as jnp
from jax import lax
from jax.experimental import pallas as pl
from jax.experimental.pallas import tpu as pltpu
from jax.experimental.pallas import tpu_sc as plsc

H = 100
L = 16


def kernel(input_ids, table):
    B, S = input_ids.shape
    V, D = table.shape
    assert S == 2 * H and D == 2 * L

    ids3 = input_ids.reshape(B, 2, H).astype(jnp.int32)

    info = plsc.get_sparse_core_info()
    NC, NS = info.num_cores, info.num_subcores
    NW = NC * NS
    PB = B // NW

    mesh = plsc.VectorSubcoreMesh(core_axis_name="c", subcore_axis_name="s")

    @functools.partial(
        pl.kernel,
        mesh=mesh,
        out_type=jax.ShapeDtypeStruct((B, D), jnp.float32),
        compiler_params=pltpu.CompilerParams(use_tc_tiling_on_sc=False),
        scratch_types=[
            pltpu.VMEM((2, 2, H), jnp.int32),
            pltpu.VMEM((2, S, D), jnp.float32),
            pltpu.VMEM((PB, D), jnp.float32),
            pltpu.SemaphoreType.DMA,
            pltpu.SemaphoreType.DMA,
        ],
    )
    def _emb_mean(ids_hbm, table_hbm, out_hbm, idx_buf, rows_buf, out_stage,
                  sem0, sem1):
        wid = lax.axis_index("s") * NC + lax.axis_index("c")
        base = wid * PB
        sems = (sem0, sem1)

        def fetch(b, buf):
            pltpu.sync_copy(ids_hbm.at[base + b], idx_buf.at[buf])
            pltpu.make_async_copy(
                table_hbm.at[idx_buf.at[buf, 0]],
                rows_buf.at[buf, pl.ds(0, H)], sems[buf]).start()
            pltpu.make_async_copy(
                table_hbm.at[idx_buf.at[buf, 1]],
                rows_buf.at[buf, pl.ds(H, H)], sems[buf]).start()

        def drain(buf):
            pltpu.make_async_copy(
                table_hbm.at[idx_buf.at[buf, 0]],
                rows_buf.at[buf, pl.ds(0, H)], sems[buf]).wait()
            pltpu.make_async_copy(
                table_hbm.at[idx_buf.at[buf, 1]],
                rows_buf.at[buf, pl.ds(H, H)], sems[buf]).wait()

        def reduce(b, buf):
            def rbody(s8, accs):
                accs = list(accs)
                for j in range(8):
                    r = s8 * 8 + j
                    k = j % 4
                    accs[2 * k] = accs[2 * k] + rows_buf[buf, r, pl.ds(0, L)]
                    accs[2 * k + 1] = (
                        accs[2 * k + 1] + rows_buf[buf, r, pl.ds(L, L)])
                return tuple(accs)

            z = jnp.zeros((L,), jnp.float32)
            a = lax.fori_loop(0, S // 8, rbody, (z,) * 8)
            lo = (a[0] + a[2]) + (a[4] + a[6])
            hi = (a[1] + a[3]) + (a[5] + a[7])
            out_stage[b, pl.ds(0, L)] = lo * (1.0 / S)
            out_stage[b, pl.ds(L, L)] = hi * (1.0 / S)

        fetch(0, 0)

        def gbody(g, carry):
            fetch(2 * g + 1, 1)
            drain(0)
            reduce(2 * g, 0)

            @pl.when(2 * g + 2 < PB)
            def _():
                fetch(2 * g + 2, 0)

            drain(1)
            reduce(2 * g + 1, 1)
            return carry

        lax.fori_loop(0, PB // 2, gbody, 0)
        pltpu.sync_copy(out_stage, out_hbm.at[pl.ds(base, PB)])

    return _emb_mean(ids3, table)


# speedup vs baseline: 40.1483x; 1.6253x over previous
import functools

import jax
import jax.numpy as jnp
from jax import lax
from jax.experimental import pallas as pl
from jax.experimental.pallas import tpu as pltpu
from jax.experimental.pallas import tpu_sc as plsc

H = 100
L = 16
CH = 8


def kernel(input_ids, table):
    B, S = input_ids.shape
    V, D = table.shape
    assert S == 2 * H and D == 2 * L

    info = plsc.get_sparse_core_info()
    NC, NS = info.num_cores, info.num_subcores
    NW = NC * NS
    PB = B // NW
    NCHUNK = PB // CH

    ids4 = input_ids.reshape(B // CH, CH, 2, H).astype(jnp.int32)

    mesh = plsc.VectorSubcoreMesh(core_axis_name="c", subcore_axis_name="s")

    @functools.partial(
        pl.kernel,
        mesh=mesh,
        out_type=jax.ShapeDtypeStruct((B, D), jnp.float32),
        compiler_params=pltpu.CompilerParams(use_tc_tiling_on_sc=False),
        scratch_types=[
            pltpu.VMEM((2, CH, 2, H), jnp.int32),
            pltpu.VMEM((2, CH * S, D), jnp.float32),
            pltpu.VMEM((PB, D), jnp.float32),
            pltpu.SemaphoreType.DMA,
            pltpu.SemaphoreType.DMA,
        ],
    )
    def _emb_mean(ids_hbm, table_hbm, out_hbm, idx_buf, rows_buf, out_stage,
                  sem0, sem1):
        wid = lax.axis_index("s") * NC + lax.axis_index("c")
        base = wid * NCHUNK
        sems = (sem0, sem1)

        def gather_descs(buf):
            return [
                pltpu.make_async_copy(
                    table_hbm.at[idx_buf.at[buf, b, h]],
                    rows_buf.at[buf, pl.ds((b * 2 + h) * H, H)],
                    sems[buf])
                for b in range(CH) for h in range(2)
            ]

        def fetch(c, buf):
            pltpu.sync_copy(ids_hbm.at[base + c], idx_buf.at[buf])
            for d in gather_descs(buf):
                d.start()

        def drain(buf):
            for d in gather_descs(buf):
                d.wait()

        def reduce(c, buf):
            def bbody(b, carry):
                def rbody(s8, accs):
                    accs = list(accs)
                    for j in range(8):
                        r = b * S + s8 * 8 + j
                        k = j % 4
                        accs[2 * k] = (
                            accs[2 * k] + rows_buf[buf, r, pl.ds(0, L)])
                        accs[2 * k + 1] = (
                            accs[2 * k + 1] + rows_buf[buf, r, pl.ds(L, L)])
                    return tuple(accs)

                z = jnp.zeros((L,), jnp.float32)
                a = lax.fori_loop(0, S // 8, rbody, (z,) * 8)
                lo = (a[0] + a[2]) + (a[4] + a[6])
                hi = (a[1] + a[3]) + (a[5] + a[7])
                out_stage[c * CH + b, pl.ds(0, L)] = lo * (1.0 / S)
                out_stage[c * CH + b, pl.ds(L, L)] = hi * (1.0 / S)
                return carry

            lax.fori_loop(0, CH, bbody, 0)

        fetch(0, 0)

        def gbody(g, carry):
            fetch(2 * g + 1, 1)
            drain(0)
            reduce(2 * g, 0)

            @pl.when(2 * g + 2 < NCHUNK)
            def _():
                fetch(2 * g + 2, 0)

            drain(1)
            reduce(2 * g + 1, 1)
            return carry

        lax.fori_loop(0, NCHUNK // 2, gbody, 0)
        pltpu.sync_copy(out_stage, out_hbm.at[pl.ds(wid * PB, PB)])

    return _emb_mean(ids4, table)
